# baseline (speedup 1.0000x reference)
import jax
import jax.numpy as jnp
from jax import lax
from jax.experimental import pallas as pl
from jax.experimental.pallas import tpu as pltpu
from jax.experimental.pallas import tpu_sc as plsc
import functools

M, N, P, Q = 16, 192, 56, 56
R = M * P * Q
NC, NS, L = 2, 16, 16
NW = NC * NS
ROWS_W = R // NW
RB = 32
CHUNKS = ROWS_W // RB
NBUF = 2
VSTEPS = RB * (N // L)

_mesh = plsc.VectorSubcoreMesh(
    core_axis_name="c", subcore_axis_name="s", num_cores=NC, num_subcores=NS
)


@functools.partial(
    pl.kernel,
    out_type=(
        jax.ShapeDtypeStruct((R, N), jnp.float32),
        jax.ShapeDtypeStruct((R, N), jnp.float32),
    ),
    mesh=_mesh,
    scratch_types=[
        pltpu.VMEM((NBUF, RB, N), jnp.float32),
        pltpu.VMEM((NBUF, RB, N), jnp.float32),
        pltpu.VMEM((NBUF, RB, N), jnp.float32),
        pltpu.VMEM((NBUF, RB, N), jnp.float32),
        pltpu.VMEM((NBUF, RB, N), jnp.float32),
        pltpu.VMEM((NBUF, RB, N), jnp.float32),
        pltpu.VMEM((NBUF, RB, N), jnp.float32),
        pltpu.VMEM((NBUF, RB, N), jnp.float32),
        pltpu.SemaphoreType.DMA,
        pltpu.SemaphoreType.DMA,
        pltpu.SemaphoreType.DMA,
        pltpu.SemaphoreType.DMA,
    ],
)
def _sc_max_fusion(ar_h, br_h, cr_h, ai_h, bi_h, ci_h, or_h, oi_h,
                   ar_v, br_v, cr_v, ai_v, bi_v, ci_v, orv, oiv,
                   in_sem0, in_sem1, out_sem0, out_sem1):
    wid = lax.axis_index("s") * NC + lax.axis_index("c")
    base = wid * ROWS_W
    in_sems = (in_sem0, in_sem1)
    out_sems = (out_sem0, out_sem1)
    in_refs = (ar_v, br_v, cr_v, ai_v, bi_v, ci_v)
    in_hbm = (ar_h, br_h, cr_h, ai_h, bi_h, ci_h)

    def issue_in(k, b):
        off = pl.multiple_of(base + k * RB, RB)
        for h, v in zip(in_hbm, in_refs):
            pltpu.async_copy(h.at[pl.ds(off, RB)], v.at[b], in_sems[b])

    def wait_in(b):
        for h, v in zip(in_hbm, in_refs):
            pltpu.make_async_copy(h.at[pl.ds(0, RB)], v.at[b], in_sems[b]).wait()

    def issue_out(k, b):
        off = pl.multiple_of(base + k * RB, RB)
        pltpu.async_copy(orv.at[b], or_h.at[pl.ds(off, RB)], out_sems[b])
        pltpu.async_copy(oiv.at[b], oi_h.at[pl.ds(off, RB)], out_sems[b])

    def wait_out(b):
        pltpu.make_async_copy(orv.at[b], or_h.at[pl.ds(0, RB)], out_sems[b]).wait()
        pltpu.make_async_copy(oiv.at[b], oi_h.at[pl.ds(0, RB)], out_sems[b]).wait()

    issue_in(0, 0)
    issue_in(1, 1)

    def step(i, _):
        for b in range(NBUF):
            k = i * NBUF + b
            wait_in(b)

            @pl.when(k >= NBUF)
            def _():
                wait_out(b)

            @plsc.parallel_loop(0, VSTEPS, unroll=8)
            def _(j):
                row = j // (N // L)
                col = (j % (N // L)) * L
                s = pl.ds(col, L)
                ra = ar_v[b, row, s]
                ia = ai_v[b, row, s]
                rb = br_v[b, row, s]
                ib = bi_v[b, row, s]
                rc = cr_v[b, row, s]
                ic = ci_v[b, row, s]
                ma = ra * ra + ia * ia
                mb = rb * rb + ib * ib
                mc = rc * rc + ic * ic
                b_wins = mb > ma
                r1 = jnp.where(b_wins, rb, ra)
                i1 = jnp.where(b_wins, ib, ia)
                m1 = jnp.maximum(ma, mb)
                c_wins = mc > m1
                orv[b, row, s] = jnp.where(c_wins, rc, r1)
                oiv[b, row, s] = jnp.where(c_wins, ic, i1)

            @pl.when(k + NBUF < CHUNKS)
            def _():
                issue_in(k + NBUF, b)

            issue_out(k, b)
        return 0

    lax.fori_loop(0, CHUNKS // NBUF, step, 0)
    k = CHUNKS - 1
    wait_in(0)
    wait_out(0)

    @plsc.parallel_loop(0, VSTEPS, unroll=8)
    def _(j):
        row = j // (N // L)
        s = pl.ds((j % (N // L)) * L, L)
        ra = ar_v[0, row, s]
        ia = ai_v[0, row, s]
        rb = br_v[0, row, s]
        ib = bi_v[0, row, s]
        rc = cr_v[0, row, s]
        ic = ci_v[0, row, s]
        ma = ra * ra + ia * ia
        mb = rb * rb + ib * ib
        mc = rc * rc + ic * ic
        b_wins = mb > ma
        r1 = jnp.where(b_wins, rb, ra)
        i1 = jnp.where(b_wins, ib, ia)
        m1 = jnp.maximum(ma, mb)
        c_wins = mc > m1
        orv[0, row, s] = jnp.where(c_wins, rc, r1)
        oiv[0, row, s] = jnp.where(c_wins, ic, i1)

    issue_out(k, 0)
    wait_out(0)
    wait_out(1)


def kernel(Fea_A_r, Fea_B_r, Fea_C_r, Fea_A_i, Fea_B_i, Fea_C_i):
    t = lambda x: x.transpose(0, 2, 3, 1).reshape(R, N)
    out_r, out_i = _sc_max_fusion(
        t(Fea_A_r), t(Fea_B_r), t(Fea_C_r),
        t(Fea_A_i), t(Fea_B_i), t(Fea_C_i),
    )
    u = lambda x: x.reshape(M, P, Q, N).transpose(0, 3, 1, 2)
    return u(out_r), u(out_i)


# speedup vs baseline: 1.0096x; 1.0096x over previous
import jax
import jax.numpy as jnp
from jax import lax
from jax.experimental import pallas as pl
from jax.experimental.pallas import tpu as pltpu
from jax.experimental.pallas import tpu_sc as plsc
import functools

M, N, P, Q = 16, 192, 56, 56
R = M * P * Q
NC, NS, L = 2, 16, 16
NW = NC * NS
ROWS_W = R // NW
RB = 32
CHUNKS = ROWS_W // RB
NBUF = 2
VSTEPS = RB * (N // L)

_mesh = plsc.VectorSubcoreMesh(
    core_axis_name="c", subcore_axis_name="s", num_cores=NC, num_subcores=NS
)


@functools.partial(
    pl.kernel,
    out_type=(
        jax.ShapeDtypeStruct((R, N), jnp.float32),
        jax.ShapeDtypeStruct((R, N), jnp.float32),
    ),
    mesh=_mesh,
    scratch_types=[
        pltpu.VMEM((NBUF, RB, N), jnp.float32),
        pltpu.VMEM((NBUF, RB, N), jnp.float32),
        pltpu.VMEM((NBUF, RB, N), jnp.float32),
        pltpu.VMEM((NBUF, RB, N), jnp.float32),
        pltpu.VMEM((NBUF, RB, N), jnp.float32),
        pltpu.VMEM((NBUF, RB, N), jnp.float32),
        pltpu.VMEM((NBUF, RB, N), jnp.float32),
        pltpu.VMEM((NBUF, RB, N), jnp.float32),
        pltpu.SemaphoreType.DMA,
        pltpu.SemaphoreType.DMA,
        pltpu.SemaphoreType.DMA,
        pltpu.SemaphoreType.DMA,
    ],
)
def _sc_max_fusion(ar_h, br_h, cr_h, ai_h, bi_h, ci_h, or_h, oi_h,
                   ar_v, br_v, cr_v, ai_v, bi_v, ci_v, orv, oiv,
                   in_sem0, in_sem1, out_sem0, out_sem1):
    wid = lax.axis_index("s") * NC + lax.axis_index("c")
    base = wid * ROWS_W
    in_sems = (in_sem0, in_sem1)
    out_sems = (out_sem0, out_sem1)
    in_refs = (ar_v, br_v, cr_v, ai_v, bi_v, ci_v)
    in_hbm = (ar_h, br_h, cr_h, ai_h, bi_h, ci_h)

    def issue_in(k, b):
        off = pl.multiple_of(base + k * RB, RB)
        for h, v in zip(in_hbm, in_refs):
            pltpu.async_copy(h.at[pl.ds(off, RB)], v.at[b], in_sems[b])

    def wait_in(b):
        for h, v in zip(in_hbm, in_refs):
            pltpu.make_async_copy(h.at[pl.ds(0, RB)], v.at[b], in_sems[b]).wait()

    def issue_out(k, b):
        off = pl.multiple_of(base + k * RB, RB)
        pltpu.async_copy(orv.at[b], or_h.at[pl.ds(off, RB)], out_sems[b])
        pltpu.async_copy(oiv.at[b], oi_h.at[pl.ds(off, RB)], out_sems[b])

    def wait_out(b):
        pltpu.make_async_copy(orv.at[b], or_h.at[pl.ds(0, RB)], out_sems[b]).wait()
        pltpu.make_async_copy(oiv.at[b], oi_h.at[pl.ds(0, RB)], out_sems[b]).wait()

    issue_in(0, 0)
    issue_in(1, 1)

    def step(i, _):
        for b in range(NBUF):
            k = i * NBUF + b
            wait_in(b)

            @pl.when(k >= NBUF)
            def _():
                wait_out(b)

            @plsc.parallel_loop(0, VSTEPS, unroll=4)
            def _(j):
                row = j // (N // L)
                col = (j % (N // L)) * L
                s = pl.ds(col, L)
                ra = ar_v[b, row, s]
                ia = ai_v[b, row, s]
                rb = br_v[b, row, s]
                ib = bi_v[b, row, s]
                rc = cr_v[b, row, s]
                ic = ci_v[b, row, s]
                ma = ra * ra + ia * ia
                mb = rb * rb + ib * ib
                mc = rc * rc + ic * ic
                b_wins = mb > ma
                r1 = jnp.where(b_wins, rb, ra)
                i1 = jnp.where(b_wins, ib, ia)
                m1 = jnp.maximum(ma, mb)
                c_wins = mc > m1
                orv[b, row, s] = jnp.where(c_wins, rc, r1)
                oiv[b, row, s] = jnp.where(c_wins, ic, i1)

            @pl.when(k + NBUF < CHUNKS)
            def _():
                issue_in(k + NBUF, b)

            issue_out(k, b)
        return 0

    lax.fori_loop(0, CHUNKS // NBUF, step, 0)
    k = CHUNKS - 1
    wait_in(0)
    wait_out(0)

    @plsc.parallel_loop(0, VSTEPS, unroll=4)
    def _(j):
        row = j // (N // L)
        s = pl.ds((j % (N // L)) * L, L)
        ra = ar_v[0, row, s]
        ia = ai_v[0, row, s]
        rb = br_v[0, row, s]
        ib = bi_v[0, row, s]
        rc = cr_v[0, row, s]
        ic = ci_v[0, row, s]
        ma = ra * ra + ia * ia
        mb = rb * rb + ib * ib
        mc = rc * rc + ic * ic
        b_wins = mb > ma
        r1 = jnp.where(b_wins, rb, ra)
        i1 = jnp.where(b_wins, ib, ia)
        m1 = jnp.maximum(ma, mb)
        c_wins = mc > m1
        orv[0, row, s] = jnp.where(c_wins, rc, r1)
        oiv[0, row, s] = jnp.where(c_wins, ic, i1)

    issue_out(k, 0)
    wait_out(0)
    wait_out(1)


def kernel(Fea_A_r, Fea_B_r, Fea_C_r, Fea_A_i, Fea_B_i, Fea_C_i):
    t = lambda x: x.transpose(0, 2, 3, 1).reshape(R, N)
    out_r, out_i = _sc_max_fusion(
        t(Fea_A_r), t(Fea_B_r), t(Fea_C_r),
        t(Fea_A_i), t(Fea_B_i), t(Fea_C_i),
    )
    u = lambda x: x.reshape(M, P, Q, N).transpose(0, 3, 1, 2)
    return u(out_r), u(out_i)
